# Initial kernel scaffold; baseline (speedup 1.0000x reference)
import jax
import jax.numpy as jnp
from jax.experimental import pallas as pl


def kernel(pos, feat, fps_idx, params):
    raise NotImplementedError("write your pallas kernel here")


# speedup vs baseline: 1.3174x; 1.3174x over previous
import functools
import math

import jax
import jax.numpy as jnp
from jax import lax
from jax.experimental import pallas as pl
from jax.experimental.pallas import tpu as pltpu
from jax.experimental.pallas import tpu_sc as plsc


def _ln(x, g, b, eps=1e-5):
    m = jnp.mean(x, axis=-1, keepdims=True)
    v = jnp.mean((x - m) ** 2, axis=-1, keepdims=True)
    return (x - m) / jnp.sqrt(v + eps) * g + b


def _fma_mm(x, w_ref, r0, k):
    acc = x[:, 0:1] * w_ref[r0:r0 + 1, :]
    for i in range(1, k):
        acc = acc + x[:, i:i + 1] * w_ref[r0 + i:r0 + i + 1, :]
    return acc


def _block_body(cs, G, d_embed, d_feat):
    T = G * cs
    inv_scale = 1.0 / math.sqrt(d_embed)

    def body(pos_ref, feat_ref, w1a_ref, w2a_ref, w1b_ref, w2b_ref,
             wqkvo_ref, vec32_ref, vecd_ref, out_ref):
        f32 = jnp.float32
        P = pos_ref[:, 0:3]
        F = feat_ref[:]

        seg_r = lax.broadcasted_iota(jnp.int32, (G, T), 1) // cs
        gid_r = lax.broadcasted_iota(jnp.int32, (G, T), 0)
        M = jnp.where(seg_r == gid_r, 1.0 / cs, 0.0).astype(f32)
        seg_c = lax.broadcasted_iota(jnp.int32, (T, G), 0) // cs
        gid_c = lax.broadcasted_iota(jnp.int32, (T, G), 1)
        Mb = jnp.where(seg_c == gid_c, 1.0, 0.0).astype(f32)

        cog = jnp.dot(M, P, preferred_element_type=f32)
        local = P - jnp.dot(Mb, cog, preferred_element_type=f32)
        n = jnp.sqrt(jnp.sum(local * local, axis=-1, keepdims=True))
        avg = jnp.dot(Mb, jnp.dot(M, local, preferred_element_type=f32),
                      preferred_element_type=f32)

        pre = (_fma_mm(local, w1a_ref, 0, 3) + n * w1a_ref[3:4, :]
               + vec32_ref[0:1, :])
        r = jax.nn.relu(_ln(pre, vec32_ref[1:2, :], vec32_ref[2:3, :]))

        pre = (jnp.dot(r, w1b_ref[0:32, :], preferred_element_type=f32)
               + jnp.dot(F, w1b_ref[32:32 + d_feat, :], preferred_element_type=f32)
               + vecd_ref[0:1, :])
        h_pos = jax.nn.relu(_ln(pre, vecd_ref[1:2, :], vecd_ref[2:3, :]))

        pre = (_fma_mm(avg, w2a_ref, 0, 3) + _fma_mm(local, w2a_ref, 3, 3)
               + vec32_ref[3:4, :])
        r_hat = jax.nn.relu(_ln(pre, vec32_ref[4:5, :], vec32_ref[5:6, :]))

        pre = (jnp.dot(r_hat, w2b_ref[0:32, :], preferred_element_type=f32)
               + jnp.dot(F, w2b_ref[32:32 + d_feat, :], preferred_element_type=f32)
               + vecd_ref[3:4, :])
        h_geo = jax.nn.relu(_ln(pre, vecd_ref[4:5, :], vecd_ref[5:6, :]))

        hp = _ln(h_pos, vecd_ref[6:7, :], vecd_ref[7:8, :])
        hg = _ln(h_geo, vecd_ref[8:9, :], vecd_ref[9:10, :])

        d = d_embed
        Q = jnp.dot(hg, wqkvo_ref[0:d, :], preferred_element_type=f32) * inv_scale
        K = jnp.dot(hg, wqkvo_ref[d:2 * d, :], preferred_element_type=f32)
        V = jnp.dot(hp, wqkvo_ref[2 * d:3 * d, :], preferred_element_type=f32)

        outs = []
        for g in range(G):
            sl = slice(g * cs, (g + 1) * cs)
            s = lax.dot_general(Q[sl, :], K[sl, :], (((1,), (1,)), ((), ())),
                                preferred_element_type=f32)
            a = jax.nn.softmax(s, axis=-1)
            outs.append(jnp.dot(a, V[sl, :], preferred_element_type=f32))
        attn = jnp.concatenate(outs, axis=0)

        out_ref[:] = (jnp.dot(attn, wqkvo_ref[3 * d:4 * d, :],
                              preferred_element_type=f32)
                      + vecd_ref[10:11, :] + h_pos)

    return body


def _run_block(pos2, feat2, bp, cs, d_embed, G):
    n_tok = pos2.shape[0]
    d_feat = feat2.shape[-1]
    pw = pos2.shape[-1]
    T = G * cs
    ngrid = n_tok // T

    wqkvo = jnp.concatenate([bp['Wq'], bp['Wk'], bp['Wv'], bp['Wo']], axis=0)
    vec32 = jnp.stack([bp['mlp_1a']['b'], bp['mlp_1a']['g'], bp['mlp_1a']['b2'],
                       bp['mlp_2a']['b'], bp['mlp_2a']['g'], bp['mlp_2a']['b2']])
    vecd = jnp.stack([bp['mlp_1b']['b'], bp['mlp_1b']['g'], bp['mlp_1b']['b2'],
                      bp['mlp_2b']['b'], bp['mlp_2b']['g'], bp['mlp_2b']['b2'],
                      bp['ln11_g'], bp['ln11_b'], bp['ln12_g'], bp['ln12_b'],
                      bp['bo']])
    weights = [bp['mlp_1a']['W'], bp['mlp_2a']['W'], bp['mlp_1b']['W'],
               bp['mlp_2b']['W'], wqkvo, vec32, vecd]

    def _full(w):
        return pl.BlockSpec(w.shape, lambda i: (0, 0))

    body = _block_body(cs, G, d_embed, d_feat)
    return pl.pallas_call(
        body,
        grid=(ngrid,),
        in_specs=[pl.BlockSpec((T, pw), lambda i: (i, 0)),
                  pl.BlockSpec((T, d_feat), lambda i: (i, 0))]
                 + [_full(w) for w in weights],
        out_specs=pl.BlockSpec((T, d_embed), lambda i: (i, 0)),
        out_shape=jax.ShapeDtypeStruct((n_tok, d_embed), jnp.float32),
        compiler_params=pltpu.CompilerParams(
            dimension_semantics=("parallel",)),
    )(pos2, feat2, *weights)


_N_DOWN = 16384
_NW = 32
_CHUNK = _N_DOWN // _NW


def _sc_gather(f1_flat, pos_flat, gidx):
    d1 = f1_flat.shape[-1]
    d2 = pos_flat.shape[-1]
    mesh = plsc.VectorSubcoreMesh(core_axis_name="c", subcore_axis_name="s")

    @functools.partial(
        pl.kernel, mesh=mesh,
        out_type=[jax.ShapeDtypeStruct((_N_DOWN, d1), jnp.float32),
                  jax.ShapeDtypeStruct((_N_DOWN, d2), jnp.float32)],
        scratch_types=[pltpu.VMEM((_CHUNK,), jnp.int32),
                       pltpu.VMEM((_CHUNK, d1), jnp.float32),
                       pltpu.SemaphoreType.DMA],
    )
    def gk(f1_hbm, pos_hbm, idx_hbm, out1_hbm, out2_hbm,
           idx_v, rows_v, sem):
        wid = lax.axis_index("s") * 2 + lax.axis_index("c")
        base = wid * _CHUNK
        pltpu.sync_copy(idx_hbm.at[pl.ds(base, _CHUNK)], idx_v)
        pltpu.async_copy(f1_hbm.at[idx_v], rows_v, sem).wait()
        pltpu.sync_copy(rows_v, out1_hbm.at[pl.ds(base, _CHUNK)])
        pltpu.async_copy(pos_hbm.at[idx_v], rows_v, sem).wait()
        pltpu.sync_copy(rows_v, out2_hbm.at[pl.ds(base, _CHUNK)])

    return gk(f1_flat, pos_flat, gidx)


def kernel(pos, feat, fps_idx, params):
    B, N, _ = pos.shape
    pos2 = pos.reshape(B * N, 3)
    feat2 = feat.reshape(B * N, feat.shape[-1])

    f1 = _run_block(pos2, feat2, params['block1'], cs=64, d_embed=128, G=8)

    pos_pad = jnp.pad(pos2, ((0, 0), (0, 125)))
    gidx = (fps_idx.astype(jnp.int32)
            + (jnp.arange(B, dtype=jnp.int32) * N)[:, None]).reshape(-1)
    f1_d, pos_d_pad = _sc_gather(f1, pos_pad, gidx)

    f2 = _run_block(pos_d_pad, f1_d, params['block2'], cs=128, d_embed=256, G=4)
    return f2.reshape(B, fps_idx.shape[1], 256)
